# Initial kernel scaffold; baseline (speedup 1.0000x reference)
import jax
import jax.numpy as jnp
from jax.experimental import pallas as pl


def kernel(image, labels, edges_nn, probas, feats0, W0, b0, W1, b1, lin1_w):
    raise NotImplementedError("write your pallas kernel here")


# speedup vs baseline: 5.0053x; 5.0053x over previous
import functools

import jax
import jax.numpy as jnp
from jax import lax
from jax.experimental import pallas as pl
from jax.experimental.pallas import tpu as pltpu
from jax.experimental.pallas import tpu_sc as plsc

N = 10000
E = 160000
EP = 163840
P = 50176
SIGMA = 0.2
NC, NS = 2, 16

N_PAD = 10240
ROWS_A = 640

_mesh = plsc.VectorSubcoreMesh(core_axis_name="c", subcore_axis_name="s")


EW_PER_W = EP // (NC * NS)
EW_CH = 128
EW_NCH = EW_PER_W // EW_CH
PX_CH = 128
PX_TOT_CH = P // PX_CH


@functools.partial(
    pl.kernel,
    out_type=(
        jax.ShapeDtypeStruct((EP,), jnp.float32),
        jax.ShapeDtypeStruct((NC * N_PAD,), jnp.float32),
        jax.ShapeDtypeStruct((NC * N_PAD,), jnp.float32),
        jax.ShapeDtypeStruct((NC * 3 * N_PAD,), jnp.float32),
    ),
    mesh=_mesh,
    scratch_types=[
        pltpu.VMEM((EW_CH,), jnp.int32),
        pltpu.VMEM((EW_CH,), jnp.int32),
        pltpu.VMEM((EW_CH,), jnp.float32),
        pltpu.VMEM((EW_CH,), jnp.float32),
        pltpu.VMEM((EW_CH,), jnp.float32),
        pltpu.VMEM((PX_CH,), jnp.int32),
        pltpu.VMEM((PX_CH,), jnp.float32),
        pltpu.VMEM((3, PX_CH), jnp.float32),
        pltpu.VMEM((ROWS_A,), jnp.float32),
        pltpu.SemaphoreType.DMA,
        pltpu.VMEM_SHARED((N_PAD,), jnp.float32),
        pltpu.VMEM_SHARED((N_PAD,), jnp.float32),
        pltpu.VMEM_SHARED((N_PAD,), jnp.float32),
        pltpu.VMEM_SHARED((N_PAD,), jnp.float32),
        pltpu.VMEM_SHARED((N_PAD,), jnp.float32),
    ],
)
def _sc_scalar(row_hbm, col_hbm, probas_hbm, lab_hbm, img_hbm,
               ew_hbm, deg_hbm, cnt_hbm, imgs_hbm,
               ridx_v, cidx_v, pr_v, pc_v, ewb_v, lidx_v, ones_v, img_v,
               zero_v, sem, deg_s, cnt_s, img0_s, img1_s, img2_s):
    img_chs = (img0_s, img1_s, img2_s)
    cid = lax.axis_index("c")
    tid = lax.axis_index("s")
    wid = cid * NS + tid

    def zb(i, _):
        zero_v[pl.ds(i * 16, 16)] = jnp.zeros((16,), jnp.float32)
        return _
    lax.fori_loop(0, ROWS_A // 16, zb, None)
    for g in range(PX_CH // 16):
        ones_v[pl.ds(g * 16, 16)] = jnp.ones((16,), jnp.float32)

    start = tid * ROWS_A
    pltpu.sync_copy(zero_v, deg_s.at[pl.ds(start, ROWS_A)])
    pltpu.sync_copy(zero_v, cnt_s.at[pl.ds(start, ROWS_A)])
    for c in range(3):
        pltpu.sync_copy(zero_v, img_chs[c].at[pl.ds(start, ROWS_A)])

    plsc.subcore_barrier()

    ebase = wid * EW_PER_W

    def edge_chunk(j, _):
        base = ebase + j * EW_CH
        pltpu.sync_copy(row_hbm.at[pl.ds(base, EW_CH)], ridx_v)
        pltpu.sync_copy(col_hbm.at[pl.ds(base, EW_CH)], cidx_v)
        pltpu.async_copy(probas_hbm.at[ridx_v], pr_v, sem).wait()
        pltpu.async_copy(probas_hbm.at[cidx_v], pc_v, sem).wait()
        for g in range(EW_CH // 16):
            pr = pr_v[pl.ds(g * 16, 16)]
            pc = pc_v[pl.ds(g * 16, 16)]
            ew16 = jnp.exp(jnp.abs(pr - pc) * (-1.0 / SIGMA))
            gidx = base + g * 16 + lax.iota(jnp.int32, 16)
            ew16 = jnp.where(gidx < E, ew16, 0.0)
            ewb_v[pl.ds(g * 16, 16)] = ew16
        pltpu.sync_copy(ewb_v, ew_hbm.at[pl.ds(base, EW_CH)])
        pltpu.sync_copy(ewb_v, deg_s.at[cidx_v], add=True)
        return _
    lax.fori_loop(0, EW_NCH, edge_chunk, None)

    base_ch = wid * (PX_TOT_CH // 32) + jnp.minimum(wid, PX_TOT_CH % 32)
    n_ch = jnp.where(wid < PX_TOT_CH % 32,
                     PX_TOT_CH // 32 + 1, PX_TOT_CH // 32)
    pbase = base_ch * PX_CH

    def px_chunk(j, _):
        base = pbase + j * PX_CH
        pltpu.sync_copy(lab_hbm.at[pl.ds(base, PX_CH)], lidx_v)
        pltpu.sync_copy(img_hbm.at[:, pl.ds(base, PX_CH)], img_v)
        pltpu.sync_copy(ones_v, cnt_s.at[lidx_v], add=True)
        for c in range(3):
            pltpu.sync_copy(img_v.at[c], img_chs[c].at[lidx_v], add=True)
        return _
    lax.fori_loop(0, n_ch, px_chunk, None)

    plsc.subcore_barrier()

    pltpu.sync_copy(deg_s.at[pl.ds(start, ROWS_A)],
                    deg_hbm.at[pl.ds(cid * N_PAD + start, ROWS_A)])
    pltpu.sync_copy(cnt_s.at[pl.ds(start, ROWS_A)],
                    cnt_hbm.at[pl.ds(cid * N_PAD + start, ROWS_A)])
    for c in range(3):
        pltpu.sync_copy(
            img_chs[c].at[pl.ds(start, ROWS_A)],
            imgs_hbm.at[pl.ds((cid * 3 + c) * N_PAD + start, ROWS_A)])


PB_CH = 128
PB_TOT_CH = P // PB_CH


def _zero_shared_rows(tid, zero2_v, acc_s):
    def zb(i, _):
        for g in range(8):
            zero2_v[i, pl.ds(g * 16, 16)] = jnp.zeros((16,), jnp.float32)
        return _
    lax.fori_loop(0, 128, zb, None)
    start = tid * ROWS_A
    for k in range(ROWS_A // 128):
        pltpu.sync_copy(zero2_v, acc_s.at[pl.ds(start + k * 128, 128), :])


def _emit_shared_rows(tid, cid, acc_s, out_hbm):
    start = tid * ROWS_A
    pltpu.sync_copy(acc_s.at[pl.ds(start, ROWS_A), :],
                    out_hbm.at[cid, pl.ds(start, ROWS_A), :])


@functools.partial(
    pl.kernel,
    out_type=jax.ShapeDtypeStruct((NC, N_PAD, 128), jnp.float32),
    mesh=_mesh,
    scratch_types=[
        pltpu.VMEM((128, 128), jnp.float32),
        pltpu.VMEM((PB_CH, 128), jnp.float32),
        pltpu.VMEM((PB_CH,), jnp.int32),
        pltpu.VMEM_SHARED((N_PAD, 128), jnp.float32),
    ],
)
def _sc_pool(feats_hbm, lab_hbm, out_hbm, zero2_v, buf_v, lidx_v, acc_s):
    cid = lax.axis_index("c")
    tid = lax.axis_index("s")

    _zero_shared_rows(tid, zero2_v, acc_s)
    plsc.subcore_barrier()

    base_ch = tid * (PB_TOT_CH // NS) + jnp.minimum(tid, PB_TOT_CH % NS)
    n_ch = jnp.where(tid < PB_TOT_CH % NS,
                     PB_TOT_CH // NS + 1, PB_TOT_CH // NS)
    pbase = base_ch * PB_CH

    def px_chunk(j, _):
        base = pbase + j * PB_CH
        pltpu.sync_copy(lab_hbm.at[pl.ds(base, PB_CH)], lidx_v)
        pltpu.sync_copy(feats_hbm.at[pl.ds(cid * P + base, PB_CH), :], buf_v)
        pltpu.sync_copy(buf_v, acc_s.at[lidx_v], add=True)
        return _
    lax.fori_loop(0, n_ch, px_chunk, None)

    plsc.subcore_barrier()
    _emit_shared_rows(tid, cid, acc_s, out_hbm)


EC_PER_T = EP // NS
EC_CH = 128
EC_NCH = EC_PER_T // EC_CH


@functools.partial(
    pl.kernel,
    out_type=jax.ShapeDtypeStruct((NC, N_PAD, 128), jnp.float32),
    mesh=_mesh,
    scratch_types=[
        pltpu.VMEM((128, 128), jnp.float32),
        pltpu.VMEM((EC_CH, 128), jnp.float32),
        pltpu.VMEM((EC_CH,), jnp.int32),
        pltpu.VMEM((EC_CH,), jnp.int32),
        pltpu.VMEM((EC_CH,), jnp.float32),
        pltpu.SemaphoreType.DMA,
        pltpu.VMEM_SHARED((N_PAD, 128), jnp.float32),
    ],
)
def _sc_edge(ys_hbm, row_hbm, col_hbm, ew_hbm, out_hbm,
             zero2_v, rows_v, ridx_v, cidx_v, ewv, sem, acc_s):
    cid = lax.axis_index("c")
    tid = lax.axis_index("s")

    _zero_shared_rows(tid, zero2_v, acc_s)
    plsc.subcore_barrier()

    ebase = tid * EC_PER_T
    tab_off = cid * N

    def edge_chunk(j, _):
        base = ebase + j * EC_CH
        pltpu.sync_copy(row_hbm.at[pl.ds(base, EC_CH)], ridx_v)
        for g in range(EC_CH // 16):
            ridx_v[pl.ds(g * 16, 16)] = ridx_v[pl.ds(g * 16, 16)] + tab_off
        pltpu.async_copy(ys_hbm.at[ridx_v], rows_v, sem).wait()
        pltpu.sync_copy(ew_hbm.at[pl.ds(base, EC_CH)], ewv)
        pltpu.sync_copy(col_hbm.at[pl.ds(base, EC_CH)], cidx_v)

        def scale(g, _c):
            ew16 = ewv[pl.ds(g * 16, 16)]
            for j in range(16):
                e = g * 16 + j
                s = ew16[j]
                for q in range(8):
                    rows_v[e, pl.ds(q * 16, 16)] = (
                        rows_v[e, pl.ds(q * 16, 16)] * s)
            return _c
        lax.fori_loop(0, EC_CH // 16, scale, None)

        pltpu.sync_copy(rows_v, acc_s.at[cidx_v], add=True)
        return _
    lax.fori_loop(0, EC_NCH, edge_chunk, None)

    plsc.subcore_barrier()
    _emit_shared_rows(tid, cid, acc_s, out_hbm)


BN = 400
GRID = N // BN


def _tc1_body(degp, cntp, imgp, w0, y1s, diag1, dinv_o, cnt_o):
    deg = degp[0] + degp[1] + 2.0
    dinv = lax.rsqrt(deg)
    cnt = cntp[0] + cntp[1]
    imgs = imgp[0] + imgp[1]
    z0 = imgs / jnp.maximum(cnt, 1.0)
    xw1 = lax.dot_general(z0, w0[...], (((1,), (0,)), ((), ())),
                          preferred_element_type=jnp.float32)
    y1 = dinv * xw1
    y1s[0] = y1[:, :128]
    y1s[1] = y1[:, 128:]
    diag1[...] = 2.0 * dinv * y1
    dinv_o[...] = dinv
    cnt_o[...] = cnt


def _tc1(deg_p, cnt_p, imgs_p, w0):
    return pl.pallas_call(
        _tc1_body,
        grid=(GRID,),
        in_specs=[
            pl.BlockSpec((NC, BN, 1), lambda i: (0, i, 0)),
            pl.BlockSpec((NC, BN, 1), lambda i: (0, i, 0)),
            pl.BlockSpec((NC, BN, 3), lambda i: (0, i, 0)),
            pl.BlockSpec((3, 256), lambda i: (0, 0)),
        ],
        out_specs=[
            pl.BlockSpec((NC, BN, 128), lambda i: (0, i, 0)),
            pl.BlockSpec((BN, 256), lambda i: (i, 0)),
            pl.BlockSpec((BN, 1), lambda i: (i, 0)),
            pl.BlockSpec((BN, 1), lambda i: (i, 0)),
        ],
        out_shape=[
            jax.ShapeDtypeStruct((NC, N, 128), jnp.float32),
            jax.ShapeDtypeStruct((N, 256), jnp.float32),
            jax.ShapeDtypeStruct((N, 1), jnp.float32),
            jax.ShapeDtypeStruct((N, 1), jnp.float32),
        ],
    )(deg_p, cnt_p, imgs_p, w0)


def _tc2_body(s1, dinv, cnt, diag1, hs, w1, b0, y2s, diag2):
    di = dinv[...]
    s1c = jnp.concatenate([s1[0], s1[1]], axis=1)
    out1 = jnp.maximum(di * s1c + diag1[...] + b0[...], 0.0)
    hc = jnp.concatenate([hs[0], hs[1]], axis=1) / jnp.maximum(cnt[...], 1.0)
    z = 0.5 * hc + 0.5 * out1
    xw2 = jnp.dot(z, w1[...], preferred_element_type=jnp.float32)
    y2 = di * xw2
    y2s[0] = y2[:, :128]
    y2s[1] = y2[:, 128:]
    diag2[...] = 2.0 * di * y2


def _tc2(s1, dinv, cnt, diag1, hs, w1, b0):
    return pl.pallas_call(
        _tc2_body,
        grid=(GRID,),
        in_specs=[
            pl.BlockSpec((NC, BN, 128), lambda i: (0, i, 0)),
            pl.BlockSpec((BN, 1), lambda i: (i, 0)),
            pl.BlockSpec((BN, 1), lambda i: (i, 0)),
            pl.BlockSpec((BN, 256), lambda i: (i, 0)),
            pl.BlockSpec((NC, BN, 128), lambda i: (0, i, 0)),
            pl.BlockSpec((256, 256), lambda i: (0, 0)),
            pl.BlockSpec((1, 256), lambda i: (0, 0)),
        ],
        out_specs=[
            pl.BlockSpec((NC, BN, 128), lambda i: (0, i, 0)),
            pl.BlockSpec((BN, 256), lambda i: (i, 0)),
        ],
        out_shape=[
            jax.ShapeDtypeStruct((NC, N, 128), jnp.float32),
            jax.ShapeDtypeStruct((N, 256), jnp.float32),
        ],
    )(s1, dinv, cnt, diag1, hs, w1, b0)


def _tc3_body(s2, dinv, diag2, b1, lw, cs, csr):
    s2c = jnp.concatenate([s2[0], s2[1]], axis=1)
    z2 = jnp.maximum(dinv[...] * s2c + diag2[...] + b1[...], 0.0)
    nrm = jnp.sqrt(jnp.sum(z2 * z2, axis=1, keepdims=True))
    r = z2 / jnp.maximum(nrm, 1e-12)
    w = lw[...]
    wn = w / jnp.maximum(
        jnp.sqrt(jnp.sum(w * w, axis=1, keepdims=True)), 1e-12)
    cs[...] = lax.dot_general(r, wn, (((1,), (1,)), ((), ())),
                              preferred_element_type=jnp.float32)
    csr[...] = r


def _tc3(s2, dinv, diag2, b1, lw):
    return pl.pallas_call(
        _tc3_body,
        grid=(GRID,),
        in_specs=[
            pl.BlockSpec((NC, BN, 128), lambda i: (0, i, 0)),
            pl.BlockSpec((BN, 1), lambda i: (i, 0)),
            pl.BlockSpec((BN, 256), lambda i: (i, 0)),
            pl.BlockSpec((1, 256), lambda i: (0, 0)),
            pl.BlockSpec((15, 256), lambda i: (0, 0)),
        ],
        out_specs=[
            pl.BlockSpec((BN, 15), lambda i: (i, 0)),
            pl.BlockSpec((BN, 256), lambda i: (i, 0)),
        ],
        out_shape=[
            jax.ShapeDtypeStruct((N, 15), jnp.float32),
            jax.ShapeDtypeStruct((N, 256), jnp.float32),
        ],
    )(s2, dinv, diag2, b1, lw)


def kernel(image, labels, edges_nn, probas, feats0, W0, b0, W1, b1, lin1_w):
    row = edges_nn[0].astype(jnp.int32)
    col = edges_nn[1].astype(jnp.int32)
    rowp = jnp.pad(row, (0, EP - E))
    colp = jnp.pad(col, (0, EP - E))
    lab = labels.reshape(P).astype(jnp.int32)
    img = image.reshape(3, P)
    feats_v = feats0.reshape(2, 128, P).transpose(0, 2, 1).reshape(2 * P, 128)

    ew, deg_f, cnt_f, imgs_f = _sc_scalar(rowp, colp, probas, lab, img)
    hs = _sc_pool(feats_v, lab)

    deg_p = deg_f.reshape(NC, N_PAD)[:, :N].reshape(NC, N, 1)
    cnt_p = cnt_f.reshape(NC, N_PAD)[:, :N].reshape(NC, N, 1)
    imgs_p = imgs_f.reshape(NC, 3, N_PAD)[:, :, :N].transpose(0, 2, 1)
    y1s, diag1, dinv, cnt = _tc1(deg_p, cnt_p, imgs_p, W0)
    s1 = _sc_edge(y1s.reshape(2 * N, 128), rowp, colp, ew)
    y2s, diag2 = _tc2(s1, dinv, cnt, diag1, hs, W1, b0.reshape(1, 256))
    s2 = _sc_edge(y2s.reshape(2 * N, 128), rowp, colp, ew)
    cs, cs_r = _tc3(s2, dinv, diag2, b1.reshape(1, 256), lin1_w)
    return (cs, cs_r)


# speedup vs baseline: 5.8811x; 1.1750x over previous
import functools

import jax
import jax.numpy as jnp
from jax import lax
from jax.experimental import pallas as pl
from jax.experimental.pallas import tpu as pltpu
from jax.experimental.pallas import tpu_sc as plsc

N = 10000
E = 160000
EP = 163840
P = 50176
SIGMA = 0.2
NC, NS = 2, 16

N_PAD = 10240
ROWS_A = 640

_mesh = plsc.VectorSubcoreMesh(core_axis_name="c", subcore_axis_name="s")


EW_PER_W = EP // (NC * NS)
EW_CH = 128
EW_NCH = EW_PER_W // EW_CH
PX_CH = 128
PX_TOT_CH = P // PX_CH


@functools.partial(
    pl.kernel,
    out_type=(
        jax.ShapeDtypeStruct((EP,), jnp.float32),
        jax.ShapeDtypeStruct((NC * N_PAD,), jnp.float32),
        jax.ShapeDtypeStruct((NC * N_PAD,), jnp.float32),
        jax.ShapeDtypeStruct((NC * 3 * N_PAD,), jnp.float32),
    ),
    mesh=_mesh,
    scratch_types=[
        pltpu.VMEM((EW_CH,), jnp.int32),
        pltpu.VMEM((EW_CH,), jnp.int32),
        pltpu.VMEM((EW_CH,), jnp.float32),
        pltpu.VMEM((EW_CH,), jnp.float32),
        pltpu.VMEM((EW_CH,), jnp.float32),
        pltpu.VMEM((PX_CH,), jnp.int32),
        pltpu.VMEM((PX_CH,), jnp.float32),
        pltpu.VMEM((3, PX_CH), jnp.float32),
        pltpu.VMEM((ROWS_A,), jnp.float32),
        pltpu.SemaphoreType.DMA,
        pltpu.VMEM_SHARED((N_PAD,), jnp.float32),
        pltpu.VMEM_SHARED((N_PAD,), jnp.float32),
        pltpu.VMEM_SHARED((N_PAD,), jnp.float32),
        pltpu.VMEM_SHARED((N_PAD,), jnp.float32),
        pltpu.VMEM_SHARED((N_PAD,), jnp.float32),
    ],
)
def _sc_scalar(row_hbm, col_hbm, probas_hbm, lab_hbm, img_hbm,
               ew_hbm, deg_hbm, cnt_hbm, imgs_hbm,
               ridx_v, cidx_v, pr_v, pc_v, ewb_v, lidx_v, ones_v, img_v,
               zero_v, sem, deg_s, cnt_s, img0_s, img1_s, img2_s):
    img_chs = (img0_s, img1_s, img2_s)
    cid = lax.axis_index("c")
    tid = lax.axis_index("s")
    wid = cid * NS + tid

    def zb(i, _):
        zero_v[pl.ds(i * 16, 16)] = jnp.zeros((16,), jnp.float32)
        return _
    lax.fori_loop(0, ROWS_A // 16, zb, None)
    for g in range(PX_CH // 16):
        ones_v[pl.ds(g * 16, 16)] = jnp.ones((16,), jnp.float32)

    start = tid * ROWS_A
    pltpu.sync_copy(zero_v, deg_s.at[pl.ds(start, ROWS_A)])
    pltpu.sync_copy(zero_v, cnt_s.at[pl.ds(start, ROWS_A)])
    for c in range(3):
        pltpu.sync_copy(zero_v, img_chs[c].at[pl.ds(start, ROWS_A)])

    plsc.subcore_barrier()

    ebase = wid * EW_PER_W

    def edge_chunk(j, _):
        base = ebase + j * EW_CH
        pltpu.sync_copy(row_hbm.at[pl.ds(base, EW_CH)], ridx_v)
        pltpu.sync_copy(col_hbm.at[pl.ds(base, EW_CH)], cidx_v)
        pltpu.async_copy(probas_hbm.at[ridx_v], pr_v, sem).wait()
        pltpu.async_copy(probas_hbm.at[cidx_v], pc_v, sem).wait()
        for g in range(EW_CH // 16):
            pr = pr_v[pl.ds(g * 16, 16)]
            pc = pc_v[pl.ds(g * 16, 16)]
            ew16 = jnp.exp(jnp.abs(pr - pc) * (-1.0 / SIGMA))
            gidx = base + g * 16 + lax.iota(jnp.int32, 16)
            ew16 = jnp.where(gidx < E, ew16, 0.0)
            ewb_v[pl.ds(g * 16, 16)] = ew16
        pltpu.sync_copy(ewb_v, ew_hbm.at[pl.ds(base, EW_CH)])
        pltpu.sync_copy(ewb_v, deg_s.at[cidx_v], add=True)
        return _
    lax.fori_loop(0, EW_NCH, edge_chunk, None)

    base_ch = wid * (PX_TOT_CH // 32) + jnp.minimum(wid, PX_TOT_CH % 32)
    n_ch = jnp.where(wid < PX_TOT_CH % 32,
                     PX_TOT_CH // 32 + 1, PX_TOT_CH // 32)
    pbase = base_ch * PX_CH

    def px_chunk(j, _):
        base = pbase + j * PX_CH
        pltpu.sync_copy(lab_hbm.at[pl.ds(base, PX_CH)], lidx_v)
        pltpu.sync_copy(img_hbm.at[:, pl.ds(base, PX_CH)], img_v)
        pltpu.sync_copy(ones_v, cnt_s.at[lidx_v], add=True)
        for c in range(3):
            pltpu.sync_copy(img_v.at[c], img_chs[c].at[lidx_v], add=True)
        return _
    lax.fori_loop(0, n_ch, px_chunk, None)

    plsc.subcore_barrier()

    pltpu.sync_copy(deg_s.at[pl.ds(start, ROWS_A)],
                    deg_hbm.at[pl.ds(cid * N_PAD + start, ROWS_A)])
    pltpu.sync_copy(cnt_s.at[pl.ds(start, ROWS_A)],
                    cnt_hbm.at[pl.ds(cid * N_PAD + start, ROWS_A)])
    for c in range(3):
        pltpu.sync_copy(
            img_chs[c].at[pl.ds(start, ROWS_A)],
            imgs_hbm.at[pl.ds((cid * 3 + c) * N_PAD + start, ROWS_A)])


PB_CH = 128
PB_TOT_CH = P // PB_CH


def _zero_shared_rows(tid, zero2_v, acc_s):
    def zb(i, _):
        for g in range(8):
            zero2_v[i, pl.ds(g * 16, 16)] = jnp.zeros((16,), jnp.float32)
        return _
    lax.fori_loop(0, 128, zb, None)
    start = tid * ROWS_A
    for k in range(ROWS_A // 128):
        pltpu.sync_copy(zero2_v, acc_s.at[pl.ds(start + k * 128, 128), :])


def _emit_shared_rows(tid, cid, acc_s, out_hbm):
    start = tid * ROWS_A
    pltpu.sync_copy(acc_s.at[pl.ds(start, ROWS_A), :],
                    out_hbm.at[cid, pl.ds(start, ROWS_A), :])


@functools.partial(
    pl.kernel,
    out_type=jax.ShapeDtypeStruct((NC, N_PAD, 128), jnp.float32),
    mesh=_mesh,
    scratch_types=[
        pltpu.VMEM((128, 128), jnp.float32),
        pltpu.VMEM((PB_CH, 128), jnp.float32),
        pltpu.VMEM((PB_CH,), jnp.int32),
        pltpu.VMEM_SHARED((N_PAD, 128), jnp.float32),
    ],
)
def _sc_pool(feats_hbm, lab_hbm, out_hbm, zero2_v, buf_v, lidx_v, acc_s):
    cid = lax.axis_index("c")
    tid = lax.axis_index("s")

    _zero_shared_rows(tid, zero2_v, acc_s)
    plsc.subcore_barrier()

    base_ch = tid * (PB_TOT_CH // NS) + jnp.minimum(tid, PB_TOT_CH % NS)
    n_ch = jnp.where(tid < PB_TOT_CH % NS,
                     PB_TOT_CH // NS + 1, PB_TOT_CH // NS)
    pbase = base_ch * PB_CH

    def px_chunk(j, _):
        base = pbase + j * PB_CH
        pltpu.sync_copy(lab_hbm.at[pl.ds(base, PB_CH)], lidx_v)
        pltpu.sync_copy(feats_hbm.at[pl.ds(cid * P + base, PB_CH), :], buf_v)
        pltpu.sync_copy(buf_v, acc_s.at[lidx_v], add=True)
        return _
    lax.fori_loop(0, n_ch, px_chunk, None)

    plsc.subcore_barrier()
    _emit_shared_rows(tid, cid, acc_s, out_hbm)


EC_PER_T = EP // NS
EC_CH = 128
EC_NCH = EC_PER_T // EC_CH


@functools.partial(
    pl.kernel,
    out_type=jax.ShapeDtypeStruct((NC, N_PAD, 128), jnp.float32),
    mesh=_mesh,
    scratch_types=[
        pltpu.VMEM((EC_CH, 128), jnp.float32),
        pltpu.VMEM((EC_CH, 128), jnp.float32),
        pltpu.VMEM((2, EC_CH), jnp.int32),
        pltpu.VMEM((2, EC_CH), jnp.int32),
        pltpu.VMEM((EC_CH,), jnp.int32),
        pltpu.VMEM((EC_CH,), jnp.int32),
        pltpu.VMEM((EC_CH,), jnp.int32),
        pltpu.VMEM((EC_CH,), jnp.int32),
        pltpu.VMEM((EC_CH,), jnp.float32),
        pltpu.VMEM((EC_CH,), jnp.float32),
        pltpu.SemaphoreType.DMA,
        pltpu.SemaphoreType.DMA,
        pltpu.SemaphoreType.DMA,
        pltpu.SemaphoreType.DMA,
        pltpu.SemaphoreType.DMA,
        pltpu.SemaphoreType.DMA,
        pltpu.VMEM_SHARED((N_PAD, 128), jnp.float32),
    ],
)
def _sc_edge(ys_hbm, rowcol_hbm, ew_hbm, out_hbm,
             rows0, rows1, idx0, idx1, ridx0, ridx1, cidx0, cidx1, ew0, ew1,
             isem0, isem1, gsem0, gsem1, ssem0, ssem1, acc_s):
    cid = lax.axis_index("c")
    tid = lax.axis_index("s")

    _zero_shared_rows(tid, rows0, acc_s)
    plsc.subcore_barrier()

    ebase = tid * EC_PER_T
    tab_off = cid * N

    rows = (rows0, rows1)
    idx = (idx0, idx1)
    ridx = (ridx0, ridx1)
    cidx = (cidx0, cidx1)
    ewv = (ew0, ew1)
    isem = (isem0, isem1)
    gsem = (gsem0, gsem1)
    ssem = (ssem0, ssem1)

    def scale_rows(p):
        def scale(g, _c):
            ew16 = ewv[p][pl.ds(g * 16, 16)]
            for jj in range(16):
                e = g * 16 + jj
                s = ew16[jj]
                for q in range(8):
                    rows[p][e, pl.ds(q * 16, 16)] = (
                        rows[p][e, pl.ds(q * 16, 16)] * s)
            return _c
        lax.fori_loop(0, EC_CH // 16, scale, None)

    def step(t, _):
        b0 = ebase + (2 * t) * EC_CH
        b1 = b0 + EC_CH
        i0a = pltpu.async_copy(rowcol_hbm.at[:, pl.ds(b0, EC_CH)],
                               idx[0], isem[0])
        i0b = pltpu.async_copy(ew_hbm.at[pl.ds(b0, EC_CH)], ewv[0], isem[0])
        i1a = pltpu.async_copy(rowcol_hbm.at[:, pl.ds(b1, EC_CH)],
                               idx[1], isem[1])
        i1b = pltpu.async_copy(ew_hbm.at[pl.ds(b1, EC_CH)], ewv[1], isem[1])
        i0a.wait(); i0b.wait()
        for g in range(EC_CH // 16):
            sl = pl.ds(g * 16, 16)
            ridx[0][sl] = idx[0][0, sl] + tab_off
            cidx[0][sl] = idx[0][1, sl]
        g0 = pltpu.async_copy(ys_hbm.at[ridx[0]], rows[0], gsem[0])
        i1a.wait(); i1b.wait()
        for g in range(EC_CH // 16):
            sl = pl.ds(g * 16, 16)
            ridx[1][sl] = idx[1][0, sl] + tab_off
            cidx[1][sl] = idx[1][1, sl]
        g1 = pltpu.async_copy(ys_hbm.at[ridx[1]], rows[1], gsem[1])
        g0.wait()
        scale_rows(0)
        pltpu.sync_copy(rows[0], acc_s.at[cidx[0]], add=True)
        g1.wait()
        scale_rows(1)
        pltpu.sync_copy(rows[1], acc_s.at[cidx[1]], add=True)
        return _
    lax.fori_loop(0, EC_NCH // 2, step, None)

    plsc.subcore_barrier()
    _emit_shared_rows(tid, cid, acc_s, out_hbm)


BN = 400
GRID = N // BN


def _tc1_body(degp, cntp, imgp, w0, y1s, diag1, dinv_o, cnt_o):
    deg = degp[0] + degp[1] + 2.0
    dinv = lax.rsqrt(deg)
    cnt = cntp[0] + cntp[1]
    imgs = imgp[0] + imgp[1]
    z0 = imgs / jnp.maximum(cnt, 1.0)
    xw1 = lax.dot_general(z0, w0[...], (((1,), (0,)), ((), ())),
                          preferred_element_type=jnp.float32)
    y1 = dinv * xw1
    y1s[0] = y1[:, :128]
    y1s[1] = y1[:, 128:]
    diag1[...] = 2.0 * dinv * y1
    dinv_o[...] = dinv
    cnt_o[...] = cnt


def _tc1(deg_p, cnt_p, imgs_p, w0):
    return pl.pallas_call(
        _tc1_body,
        grid=(GRID,),
        in_specs=[
            pl.BlockSpec((NC, BN, 1), lambda i: (0, i, 0)),
            pl.BlockSpec((NC, BN, 1), lambda i: (0, i, 0)),
            pl.BlockSpec((NC, BN, 3), lambda i: (0, i, 0)),
            pl.BlockSpec((3, 256), lambda i: (0, 0)),
        ],
        out_specs=[
            pl.BlockSpec((NC, BN, 128), lambda i: (0, i, 0)),
            pl.BlockSpec((BN, 256), lambda i: (i, 0)),
            pl.BlockSpec((BN, 1), lambda i: (i, 0)),
            pl.BlockSpec((BN, 1), lambda i: (i, 0)),
        ],
        out_shape=[
            jax.ShapeDtypeStruct((NC, N, 128), jnp.float32),
            jax.ShapeDtypeStruct((N, 256), jnp.float32),
            jax.ShapeDtypeStruct((N, 1), jnp.float32),
            jax.ShapeDtypeStruct((N, 1), jnp.float32),
        ],
    )(deg_p, cnt_p, imgs_p, w0)


def _tc2_body(s1, dinv, cnt, diag1, hs, w1, b0, y2s, diag2):
    di = dinv[...]
    s1c = jnp.concatenate([s1[0], s1[1]], axis=1)
    out1 = jnp.maximum(di * s1c + diag1[...] + b0[...], 0.0)
    hc = jnp.concatenate([hs[0], hs[1]], axis=1) / jnp.maximum(cnt[...], 1.0)
    z = 0.5 * hc + 0.5 * out1
    xw2 = jnp.dot(z, w1[...], preferred_element_type=jnp.float32)
    y2 = di * xw2
    y2s[0] = y2[:, :128]
    y2s[1] = y2[:, 128:]
    diag2[...] = 2.0 * di * y2


def _tc2(s1, dinv, cnt, diag1, hs, w1, b0):
    return pl.pallas_call(
        _tc2_body,
        grid=(GRID,),
        in_specs=[
            pl.BlockSpec((NC, BN, 128), lambda i: (0, i, 0)),
            pl.BlockSpec((BN, 1), lambda i: (i, 0)),
            pl.BlockSpec((BN, 1), lambda i: (i, 0)),
            pl.BlockSpec((BN, 256), lambda i: (i, 0)),
            pl.BlockSpec((NC, BN, 128), lambda i: (0, i, 0)),
            pl.BlockSpec((256, 256), lambda i: (0, 0)),
            pl.BlockSpec((1, 256), lambda i: (0, 0)),
        ],
        out_specs=[
            pl.BlockSpec((NC, BN, 128), lambda i: (0, i, 0)),
            pl.BlockSpec((BN, 256), lambda i: (i, 0)),
        ],
        out_shape=[
            jax.ShapeDtypeStruct((NC, N, 128), jnp.float32),
            jax.ShapeDtypeStruct((N, 256), jnp.float32),
        ],
    )(s1, dinv, cnt, diag1, hs, w1, b0)


def _tc3_body(s2, dinv, diag2, b1, lw, cs, csr):
    s2c = jnp.concatenate([s2[0], s2[1]], axis=1)
    z2 = jnp.maximum(dinv[...] * s2c + diag2[...] + b1[...], 0.0)
    nrm = jnp.sqrt(jnp.sum(z2 * z2, axis=1, keepdims=True))
    r = z2 / jnp.maximum(nrm, 1e-12)
    w = lw[...]
    wn = w / jnp.maximum(
        jnp.sqrt(jnp.sum(w * w, axis=1, keepdims=True)), 1e-12)
    cs[...] = lax.dot_general(r, wn, (((1,), (1,)), ((), ())),
                              preferred_element_type=jnp.float32)
    csr[...] = r


def _tc3(s2, dinv, diag2, b1, lw):
    return pl.pallas_call(
        _tc3_body,
        grid=(GRID,),
        in_specs=[
            pl.BlockSpec((NC, BN, 128), lambda i: (0, i, 0)),
            pl.BlockSpec((BN, 1), lambda i: (i, 0)),
            pl.BlockSpec((BN, 256), lambda i: (i, 0)),
            pl.BlockSpec((1, 256), lambda i: (0, 0)),
            pl.BlockSpec((15, 256), lambda i: (0, 0)),
        ],
        out_specs=[
            pl.BlockSpec((BN, 15), lambda i: (i, 0)),
            pl.BlockSpec((BN, 256), lambda i: (i, 0)),
        ],
        out_shape=[
            jax.ShapeDtypeStruct((N, 15), jnp.float32),
            jax.ShapeDtypeStruct((N, 256), jnp.float32),
        ],
    )(s2, dinv, diag2, b1, lw)


def kernel(image, labels, edges_nn, probas, feats0, W0, b0, W1, b1, lin1_w):
    row = edges_nn[0].astype(jnp.int32)
    col = edges_nn[1].astype(jnp.int32)
    rowp = jnp.pad(row, (0, EP - E))
    colp = jnp.pad(col, (0, EP - E))
    rowcol = jnp.stack([rowp, colp])
    lab = labels.reshape(P).astype(jnp.int32)
    img = image.reshape(3, P)
    feats_v = feats0.reshape(2, 128, P).transpose(0, 2, 1).reshape(2 * P, 128)

    ew, deg_f, cnt_f, imgs_f = _sc_scalar(rowp, colp, probas, lab, img)
    hs = _sc_pool(feats_v, lab)

    deg_p = deg_f.reshape(NC, N_PAD)[:, :N].reshape(NC, N, 1)
    cnt_p = cnt_f.reshape(NC, N_PAD)[:, :N].reshape(NC, N, 1)
    imgs_p = imgs_f.reshape(NC, 3, N_PAD)[:, :, :N].transpose(0, 2, 1)
    y1s, diag1, dinv, cnt = _tc1(deg_p, cnt_p, imgs_p, W0)
    s1 = _sc_edge(y1s.reshape(2 * N, 128), rowcol, ew)
    y2s, diag2 = _tc2(s1, dinv, cnt, diag1, hs, W1, b0.reshape(1, 256))
    s2 = _sc_edge(y2s.reshape(2 * N, 128), rowcol, ew)
    cs, cs_r = _tc3(s2, dinv, diag2, b1.reshape(1, 256), lin1_w)
    return (cs, cs_r)


# speedup vs baseline: 6.0761x; 1.0332x over previous
import functools

import jax
import jax.numpy as jnp
from jax import lax
from jax.experimental import pallas as pl
from jax.experimental.pallas import tpu as pltpu
from jax.experimental.pallas import tpu_sc as plsc

N = 10000
E = 160000
EP = 163840
P = 50176
SIGMA = 0.2
NC, NS = 2, 16

N_PAD = 10240
ROWS_A = 640

_mesh = plsc.VectorSubcoreMesh(core_axis_name="c", subcore_axis_name="s")


EW_PER_W = EP // (NC * NS)
EW_CH = 128
EW_NCH = EW_PER_W // EW_CH
PX_CH = 128
PX_TOT_CH = P // PX_CH


@functools.partial(
    pl.kernel,
    out_type=(
        jax.ShapeDtypeStruct((EP,), jnp.float32),
        jax.ShapeDtypeStruct((NC * N_PAD,), jnp.float32),
        jax.ShapeDtypeStruct((NC * N_PAD,), jnp.float32),
        jax.ShapeDtypeStruct((NC * 3 * N_PAD,), jnp.float32),
    ),
    mesh=_mesh,
    scratch_types=[
        pltpu.VMEM((EW_CH,), jnp.int32),
        pltpu.VMEM((EW_CH,), jnp.int32),
        pltpu.VMEM((EW_CH,), jnp.float32),
        pltpu.VMEM((EW_CH,), jnp.float32),
        pltpu.VMEM((EW_CH,), jnp.float32),
        pltpu.VMEM((PX_CH,), jnp.int32),
        pltpu.VMEM((PX_CH,), jnp.float32),
        pltpu.VMEM((3, PX_CH), jnp.float32),
        pltpu.VMEM((ROWS_A,), jnp.float32),
        pltpu.SemaphoreType.DMA,
        pltpu.VMEM_SHARED((N_PAD,), jnp.float32),
        pltpu.VMEM_SHARED((N_PAD,), jnp.float32),
        pltpu.VMEM_SHARED((N_PAD,), jnp.float32),
        pltpu.VMEM_SHARED((N_PAD,), jnp.float32),
        pltpu.VMEM_SHARED((N_PAD,), jnp.float32),
    ],
)
def _sc_scalar(row_hbm, col_hbm, probas_hbm, lab_hbm, img_hbm,
               ew_hbm, deg_hbm, cnt_hbm, imgs_hbm,
               ridx_v, cidx_v, pr_v, pc_v, ewb_v, lidx_v, ones_v, img_v,
               zero_v, sem, deg_s, cnt_s, img0_s, img1_s, img2_s):
    img_chs = (img0_s, img1_s, img2_s)
    cid = lax.axis_index("c")
    tid = lax.axis_index("s")
    wid = cid * NS + tid

    def zb(i, _):
        zero_v[pl.ds(i * 16, 16)] = jnp.zeros((16,), jnp.float32)
        return _
    lax.fori_loop(0, ROWS_A // 16, zb, None)
    for g in range(PX_CH // 16):
        ones_v[pl.ds(g * 16, 16)] = jnp.ones((16,), jnp.float32)

    start = tid * ROWS_A
    pltpu.sync_copy(zero_v, deg_s.at[pl.ds(start, ROWS_A)])
    pltpu.sync_copy(zero_v, cnt_s.at[pl.ds(start, ROWS_A)])
    for c in range(3):
        pltpu.sync_copy(zero_v, img_chs[c].at[pl.ds(start, ROWS_A)])

    plsc.subcore_barrier()

    ebase = wid * EW_PER_W

    def edge_chunk(j, _):
        base = ebase + j * EW_CH
        pltpu.sync_copy(row_hbm.at[pl.ds(base, EW_CH)], ridx_v)
        pltpu.sync_copy(col_hbm.at[pl.ds(base, EW_CH)], cidx_v)
        pltpu.async_copy(probas_hbm.at[ridx_v], pr_v, sem).wait()
        pltpu.async_copy(probas_hbm.at[cidx_v], pc_v, sem).wait()
        for g in range(EW_CH // 16):
            pr = pr_v[pl.ds(g * 16, 16)]
            pc = pc_v[pl.ds(g * 16, 16)]
            ew16 = jnp.exp(jnp.abs(pr - pc) * (-1.0 / SIGMA))
            gidx = base + g * 16 + lax.iota(jnp.int32, 16)
            ew16 = jnp.where(gidx < E, ew16, 0.0)
            ewb_v[pl.ds(g * 16, 16)] = ew16
        pltpu.sync_copy(ewb_v, ew_hbm.at[pl.ds(base, EW_CH)])
        pltpu.sync_copy(ewb_v, deg_s.at[cidx_v], add=True)
        return _
    lax.fori_loop(0, EW_NCH, edge_chunk, None)

    base_ch = wid * (PX_TOT_CH // 32) + jnp.minimum(wid, PX_TOT_CH % 32)
    n_ch = jnp.where(wid < PX_TOT_CH % 32,
                     PX_TOT_CH // 32 + 1, PX_TOT_CH // 32)
    pbase = base_ch * PX_CH

    def px_chunk(j, _):
        base = pbase + j * PX_CH
        pltpu.sync_copy(lab_hbm.at[pl.ds(base, PX_CH)], lidx_v)
        pltpu.sync_copy(img_hbm.at[:, pl.ds(base, PX_CH)], img_v)
        pltpu.sync_copy(ones_v, cnt_s.at[lidx_v], add=True)
        for c in range(3):
            pltpu.sync_copy(img_v.at[c], img_chs[c].at[lidx_v], add=True)
        return _
    lax.fori_loop(0, n_ch, px_chunk, None)

    plsc.subcore_barrier()

    pltpu.sync_copy(deg_s.at[pl.ds(start, ROWS_A)],
                    deg_hbm.at[pl.ds(cid * N_PAD + start, ROWS_A)])
    pltpu.sync_copy(cnt_s.at[pl.ds(start, ROWS_A)],
                    cnt_hbm.at[pl.ds(cid * N_PAD + start, ROWS_A)])
    for c in range(3):
        pltpu.sync_copy(
            img_chs[c].at[pl.ds(start, ROWS_A)],
            imgs_hbm.at[pl.ds((cid * 3 + c) * N_PAD + start, ROWS_A)])


PB_CH = 128
PB_TOT_CH = P // PB_CH


def _zero_shared_rows(tid, zero2_v, acc_s):
    def zb(i, _):
        for g in range(8):
            zero2_v[i, pl.ds(g * 16, 16)] = jnp.zeros((16,), jnp.float32)
        return _
    lax.fori_loop(0, 128, zb, None)
    start = tid * ROWS_A
    for k in range(ROWS_A // 128):
        pltpu.sync_copy(zero2_v, acc_s.at[pl.ds(start + k * 128, 128), :])


def _emit_shared_rows(tid, cid, acc_s, out_hbm):
    start = tid * ROWS_A
    pltpu.sync_copy(acc_s.at[pl.ds(start, ROWS_A), :],
                    out_hbm.at[cid, pl.ds(start, ROWS_A), :])


@functools.partial(
    pl.kernel,
    out_type=jax.ShapeDtypeStruct((NC, N_PAD, 128), jnp.float32),
    mesh=_mesh,
    scratch_types=[
        pltpu.VMEM((128, 128), jnp.float32),
        pltpu.VMEM((PB_CH, 128), jnp.float32),
        pltpu.VMEM((PB_CH,), jnp.int32),
        pltpu.VMEM_SHARED((N_PAD, 128), jnp.float32),
    ],
)
def _sc_pool(feats_hbm, lab_hbm, out_hbm, zero2_v, buf_v, lidx_v, acc_s):
    cid = lax.axis_index("c")
    tid = lax.axis_index("s")

    _zero_shared_rows(tid, zero2_v, acc_s)
    plsc.subcore_barrier()

    base_ch = tid * (PB_TOT_CH // NS) + jnp.minimum(tid, PB_TOT_CH % NS)
    n_ch = jnp.where(tid < PB_TOT_CH % NS,
                     PB_TOT_CH // NS + 1, PB_TOT_CH // NS)
    pbase = base_ch * PB_CH

    def px_chunk(j, _):
        base = pbase + j * PB_CH
        pltpu.sync_copy(lab_hbm.at[pl.ds(base, PB_CH)], lidx_v)
        pltpu.sync_copy(feats_hbm.at[pl.ds(cid * P + base, PB_CH), :], buf_v)
        pltpu.sync_copy(buf_v, acc_s.at[lidx_v], add=True)
        return _
    lax.fori_loop(0, n_ch, px_chunk, None)

    plsc.subcore_barrier()
    _emit_shared_rows(tid, cid, acc_s, out_hbm)


EC_PER_T = EP // NS
EC_CH = 128
EC_NCH = EC_PER_T // EC_CH


@functools.partial(
    pl.kernel,
    out_type=jax.ShapeDtypeStruct((NC, N_PAD, 128), jnp.float32),
    mesh=_mesh,
    scratch_types=[
        pltpu.VMEM((EC_CH, 128), jnp.float32),
        pltpu.VMEM((EC_CH, 128), jnp.float32),
        pltpu.VMEM((2, EC_CH), jnp.int32),
        pltpu.VMEM((2, EC_CH), jnp.int32),
        pltpu.VMEM((EC_CH,), jnp.int32),
        pltpu.VMEM((EC_CH,), jnp.int32),
        pltpu.VMEM((EC_CH,), jnp.int32),
        pltpu.VMEM((EC_CH,), jnp.int32),
        pltpu.VMEM((EC_CH,), jnp.float32),
        pltpu.VMEM((EC_CH,), jnp.float32),
        pltpu.SemaphoreType.DMA,
        pltpu.SemaphoreType.DMA,
        pltpu.SemaphoreType.DMA,
        pltpu.SemaphoreType.DMA,
        pltpu.SemaphoreType.DMA,
        pltpu.SemaphoreType.DMA,
        pltpu.VMEM_SHARED((N_PAD, 128), jnp.float32),
    ],
)
def _sc_edge(ys_hbm, rowcol_hbm, ew_hbm, out_hbm,
             rows0, rows1, idx0, idx1, ridx0, ridx1, cidx0, cidx1, ew0, ew1,
             isem0, isem1, gsem0, gsem1, ssem0, ssem1, acc_s):
    cid = lax.axis_index("c")
    tid = lax.axis_index("s")

    _zero_shared_rows(tid, rows0, acc_s)
    plsc.subcore_barrier()

    ebase = tid * EC_PER_T
    tab_off = cid * N

    rows = (rows0, rows1)
    idx = (idx0, idx1)
    ridx = (ridx0, ridx1)
    cidx = (cidx0, cidx1)
    ewv = (ew0, ew1)
    isem = (isem0, isem1)
    gsem = (gsem0, gsem1)
    ssem = (ssem0, ssem1)

    def scale_rows(p):
        def scale(g, _c):
            ew16 = ewv[p][pl.ds(g * 16, 16)]
            for jj in range(16):
                e = g * 16 + jj
                s = ew16[jj]
                for q in range(8):
                    rows[p][e, pl.ds(q * 16, 16)] = (
                        rows[p][e, pl.ds(q * 16, 16)] * s)
            return _c
        lax.fori_loop(0, EC_CH // 16, scale, None)

    def step(t, _):
        b0 = ebase + (2 * t) * EC_CH
        b1 = b0 + EC_CH
        i0a = pltpu.async_copy(rowcol_hbm.at[:, pl.ds(b0, EC_CH)],
                               idx[0], isem[0])
        i0b = pltpu.async_copy(ew_hbm.at[pl.ds(b0, EC_CH)], ewv[0], isem[0])
        i1a = pltpu.async_copy(rowcol_hbm.at[:, pl.ds(b1, EC_CH)],
                               idx[1], isem[1])
        i1b = pltpu.async_copy(ew_hbm.at[pl.ds(b1, EC_CH)], ewv[1], isem[1])
        i0a.wait(); i0b.wait()
        for g in range(EC_CH // 16):
            sl = pl.ds(g * 16, 16)
            ridx[0][sl] = idx[0][0, sl] + tab_off
            cidx[0][sl] = idx[0][1, sl]
        g0 = pltpu.async_copy(ys_hbm.at[ridx[0]], rows[0], gsem[0])
        i1a.wait(); i1b.wait()
        for g in range(EC_CH // 16):
            sl = pl.ds(g * 16, 16)
            ridx[1][sl] = idx[1][0, sl] + tab_off
            cidx[1][sl] = idx[1][1, sl]
        g1 = pltpu.async_copy(ys_hbm.at[ridx[1]], rows[1], gsem[1])
        g0.wait()
        scale_rows(0)
        s0 = pltpu.async_copy(rows[0], acc_s.at[cidx[0]], ssem[0], add=True)
        g1.wait()
        scale_rows(1)
        s1 = pltpu.async_copy(rows[1], acc_s.at[cidx[1]], ssem[1], add=True)
        s0.wait()
        s1.wait()
        return _
    lax.fori_loop(0, EC_NCH // 2, step, None)

    plsc.subcore_barrier()
    _emit_shared_rows(tid, cid, acc_s, out_hbm)


BN = 400
GRID = N // BN


def _tc1_body(degp, cntp, imgp, w0, y1s, diag1, dinv_o, cnt_o):
    deg = degp[0] + degp[1] + 2.0
    dinv = lax.rsqrt(deg)
    cnt = cntp[0] + cntp[1]
    imgs = imgp[0] + imgp[1]
    z0 = imgs / jnp.maximum(cnt, 1.0)
    xw1 = lax.dot_general(z0, w0[...], (((1,), (0,)), ((), ())),
                          preferred_element_type=jnp.float32)
    y1 = dinv * xw1
    y1s[0] = y1[:, :128]
    y1s[1] = y1[:, 128:]
    diag1[...] = 2.0 * dinv * y1
    dinv_o[...] = dinv
    cnt_o[...] = cnt


def _tc1(deg_p, cnt_p, imgs_p, w0):
    return pl.pallas_call(
        _tc1_body,
        grid=(GRID,),
        in_specs=[
            pl.BlockSpec((NC, BN, 1), lambda i: (0, i, 0)),
            pl.BlockSpec((NC, BN, 1), lambda i: (0, i, 0)),
            pl.BlockSpec((NC, BN, 3), lambda i: (0, i, 0)),
            pl.BlockSpec((3, 256), lambda i: (0, 0)),
        ],
        out_specs=[
            pl.BlockSpec((NC, BN, 128), lambda i: (0, i, 0)),
            pl.BlockSpec((BN, 256), lambda i: (i, 0)),
            pl.BlockSpec((BN, 1), lambda i: (i, 0)),
            pl.BlockSpec((BN, 1), lambda i: (i, 0)),
        ],
        out_shape=[
            jax.ShapeDtypeStruct((NC, N, 128), jnp.float32),
            jax.ShapeDtypeStruct((N, 256), jnp.float32),
            jax.ShapeDtypeStruct((N, 1), jnp.float32),
            jax.ShapeDtypeStruct((N, 1), jnp.float32),
        ],
    )(deg_p, cnt_p, imgs_p, w0)


def _tc2_body(s1, dinv, cnt, diag1, hs, w1, b0, y2s, diag2):
    di = dinv[...]
    s1c = jnp.concatenate([s1[0], s1[1]], axis=1)
    out1 = jnp.maximum(di * s1c + diag1[...] + b0[...], 0.0)
    hc = jnp.concatenate([hs[0], hs[1]], axis=1) / jnp.maximum(cnt[...], 1.0)
    z = 0.5 * hc + 0.5 * out1
    xw2 = jnp.dot(z, w1[...], preferred_element_type=jnp.float32)
    y2 = di * xw2
    y2s[0] = y2[:, :128]
    y2s[1] = y2[:, 128:]
    diag2[...] = 2.0 * di * y2


def _tc2(s1, dinv, cnt, diag1, hs, w1, b0):
    return pl.pallas_call(
        _tc2_body,
        grid=(GRID,),
        in_specs=[
            pl.BlockSpec((NC, BN, 128), lambda i: (0, i, 0)),
            pl.BlockSpec((BN, 1), lambda i: (i, 0)),
            pl.BlockSpec((BN, 1), lambda i: (i, 0)),
            pl.BlockSpec((BN, 256), lambda i: (i, 0)),
            pl.BlockSpec((NC, BN, 128), lambda i: (0, i, 0)),
            pl.BlockSpec((256, 256), lambda i: (0, 0)),
            pl.BlockSpec((1, 256), lambda i: (0, 0)),
        ],
        out_specs=[
            pl.BlockSpec((NC, BN, 128), lambda i: (0, i, 0)),
            pl.BlockSpec((BN, 256), lambda i: (i, 0)),
        ],
        out_shape=[
            jax.ShapeDtypeStruct((NC, N, 128), jnp.float32),
            jax.ShapeDtypeStruct((N, 256), jnp.float32),
        ],
    )(s1, dinv, cnt, diag1, hs, w1, b0)


def _tc3_body(s2, dinv, diag2, b1, lw, cs, csr):
    s2c = jnp.concatenate([s2[0], s2[1]], axis=1)
    z2 = jnp.maximum(dinv[...] * s2c + diag2[...] + b1[...], 0.0)
    nrm = jnp.sqrt(jnp.sum(z2 * z2, axis=1, keepdims=True))
    r = z2 / jnp.maximum(nrm, 1e-12)
    w = lw[...]
    wn = w / jnp.maximum(
        jnp.sqrt(jnp.sum(w * w, axis=1, keepdims=True)), 1e-12)
    cs[...] = lax.dot_general(r, wn, (((1,), (1,)), ((), ())),
                              preferred_element_type=jnp.float32)
    csr[...] = r


def _tc3(s2, dinv, diag2, b1, lw):
    return pl.pallas_call(
        _tc3_body,
        grid=(GRID,),
        in_specs=[
            pl.BlockSpec((NC, BN, 128), lambda i: (0, i, 0)),
            pl.BlockSpec((BN, 1), lambda i: (i, 0)),
            pl.BlockSpec((BN, 256), lambda i: (i, 0)),
            pl.BlockSpec((1, 256), lambda i: (0, 0)),
            pl.BlockSpec((15, 256), lambda i: (0, 0)),
        ],
        out_specs=[
            pl.BlockSpec((BN, 15), lambda i: (i, 0)),
            pl.BlockSpec((BN, 256), lambda i: (i, 0)),
        ],
        out_shape=[
            jax.ShapeDtypeStruct((N, 15), jnp.float32),
            jax.ShapeDtypeStruct((N, 256), jnp.float32),
        ],
    )(s2, dinv, diag2, b1, lw)


def kernel(image, labels, edges_nn, probas, feats0, W0, b0, W1, b1, lin1_w):
    row = edges_nn[0].astype(jnp.int32)
    col = edges_nn[1].astype(jnp.int32)
    rowp = jnp.pad(row, (0, EP - E))
    colp = jnp.pad(col, (0, EP - E))
    rowcol = jnp.stack([rowp, colp])
    lab = labels.reshape(P).astype(jnp.int32)
    img = image.reshape(3, P)
    feats_v = feats0.reshape(2, 128, P).transpose(0, 2, 1).reshape(2 * P, 128)

    ew, deg_f, cnt_f, imgs_f = _sc_scalar(rowp, colp, probas, lab, img)
    hs = _sc_pool(feats_v, lab)

    deg_p = deg_f.reshape(NC, N_PAD)[:, :N].reshape(NC, N, 1)
    cnt_p = cnt_f.reshape(NC, N_PAD)[:, :N].reshape(NC, N, 1)
    imgs_p = imgs_f.reshape(NC, 3, N_PAD)[:, :, :N].transpose(0, 2, 1)
    y1s, diag1, dinv, cnt = _tc1(deg_p, cnt_p, imgs_p, W0)
    s1 = _sc_edge(y1s.reshape(2 * N, 128), rowcol, ew)
    y2s, diag2 = _tc2(s1, dinv, cnt, diag1, hs, W1, b0.reshape(1, 256))
    s2 = _sc_edge(y2s.reshape(2 * N, 128), rowcol, ew)
    cs, cs_r = _tc3(s2, dinv, diag2, b1.reshape(1, 256), lin1_w)
    return (cs, cs_r)
